# baseline (speedup 1.0000x reference)
import functools

import jax
import jax.numpy as jnp
from jax import lax
from jax.experimental import pallas as pl
from jax.experimental.pallas import tpu as pltpu
from jax.experimental.pallas import tpu_sc as plsc

NC, NS = 2, 16
NW = NC * NS
B = 16384
D = 32
GRP = 128 // D
BPW = B // NW
CHUNK = 128
NCHUNK = BPW // CHUNK
RL = 8192
S = 131072
NB = S // RL


def _bf16_top(x):
    xi = lax.bitcast_convert_type(x, jnp.uint32)
    r = xi + jnp.uint32(0x7FFF) + ((xi >> 16) & jnp.uint32(1))
    return r & jnp.uint32(0xFFFF0000)


def _relayout_body(*refs):
    uo_ref, io_ref = refs[16], refs[17]
    for out_ref, blocks in ((uo_ref, refs[0:8]), (io_ref, refs[8:16])):
        lo = jnp.concatenate([b[...] for b in blocks[:4]], axis=0)
        hi = jnp.concatenate([b[...] for b in blocks[4:]], axis=0)
        w = _bf16_top(lo) | (_bf16_top(hi) >> 16)
        out_ref[...] = lax.bitcast_convert_type(w.T, jnp.float32)


def _tc_relayout(ut_t, it_t):
    nvb = (ut_t.shape[1] + RL - 1) // RL
    specs = [pl.BlockSpec(
                 (32, RL),
                 (lambda p: (lambda b: (0, jnp.minimum(p * NB + b, nvb - 1))))(p))
             for p in range(8)]
    mesh = pltpu.create_tensorcore_mesh("tc", num_cores=1)

    @functools.partial(
        pl.kernel,
        mesh=mesh,
        out_type=[jax.ShapeDtypeStruct((S, 128), jnp.float32),
                  jax.ShapeDtypeStruct((S, 128), jnp.float32)],
    )
    def relayout_kernel(ut_hbm, it_hbm, uo_hbm, io_hbm):
        pltpu.emit_pipeline(
            _relayout_body,
            grid=(NB,),
            in_specs=[pl.BlockSpec((32, RL), s.index_map) for s in specs] * 2,
            out_specs=[pl.BlockSpec((RL, 128), lambda b: (b, 0)),
                       pl.BlockSpec((RL, 128), lambda b: (b, 0))],
            core_axis_name="tc",
            dimension_semantics=(pltpu.PARALLEL,),
        )(*([ut_hbm] * 8 + [it_hbm] * 8 + [uo_hbm, io_hbm]))

    return relayout_kernel(ut_t, it_t)


def _sc_gather(ut4, it4, u_idx4, i_idx4):
    mesh = plsc.VectorSubcoreMesh(core_axis_name="c", subcore_axis_name="s")

    @functools.partial(
        pl.kernel,
        mesh=mesh,
        out_type=[
            jax.ShapeDtypeStruct((B, 128), jnp.float32),
            jax.ShapeDtypeStruct((B, 128), jnp.float32),
        ],
        scratch_types=[
            pltpu.VMEM((NCHUNK, CHUNK), jnp.int32),
            pltpu.VMEM((NCHUNK, CHUNK), jnp.int32),
            pltpu.VMEM((4, CHUNK, 128), jnp.float32),
            pltpu.SemaphoreType.DMA,
            pltpu.SemaphoreType.DMA,
        ],
    )
    def gather_kernel(ut_hbm, it_hbm, ui_hbm, ii_hbm, u_out, i_out,
                      uidx_v, iidx_v, bufs, gsem, wsem):
        wid = lax.axis_index("s") * NC + lax.axis_index("c")
        base = wid * BPW
        pltpu.sync_copy(ui_hbm.at[wid], uidx_v)
        pltpu.sync_copy(ii_hbm.at[wid], iidx_v)
        for tab, idx, out in ((ut_hbm, uidx_v, u_out), (it_hbm, iidx_v, i_out)):
            gds = [pltpu.async_copy(tab.at[idx.at[c]], bufs.at[c], gsem)
                   for c in range(NCHUNK)]
            for g in gds:
                g.wait()
            wbs = [pltpu.async_copy(
                       bufs.at[c], out.at[pl.ds(base + c * CHUNK, CHUNK)], wsem)
                   for c in range(NCHUNK)]
            for w in wbs:
                w.wait()

    return gather_kernel(ut4, it4, u_idx4, i_idx4)


def _mlp_body(uw_ref, iw_ref, ur_ref, ir_ref,
              w1a_ref, w1b_ref, b1_ref, w2_ref, b2_ref, o_ref):
    blk = uw_ref.shape[0]
    ones = jnp.ones((1, 128), jnp.float32)
    urb = lax.dot_general(ur_ref[...], ones, (((1,), (0,)), ((), ())),
                          preferred_element_type=jnp.float32)
    irb = lax.dot_general(ir_ref[...], ones, (((1,), (0,)), ((), ())),
                          preferred_element_type=jnp.float32)
    group = (lax.broadcasted_iota(jnp.int32, (blk, 128), 1) // D
             ).astype(jnp.float32)
    uw = lax.bitcast_convert_type(uw_ref[...], jnp.uint32)
    iw = lax.bitcast_convert_type(iw_ref[...], jnp.uint32)
    uv = lax.bitcast_convert_type(
        jnp.where(urb >= 4.0, uw << 16, uw & jnp.uint32(0xFFFF0000)),
        jnp.float32)
    iv = lax.bitcast_convert_type(
        jnp.where(irb >= 4.0, iw << 16, iw & jnp.uint32(0xFFFF0000)),
        jnp.float32)
    upm = urb - 4.0 * jnp.floor(urb * 0.25)
    ipm = irb - 4.0 * jnp.floor(irb * 0.25)
    um = jnp.where(upm == group, uv, 0.0)
    im = jnp.where(ipm == group, iv, 0.0)
    h = lax.dot_general(um, w1a_ref[...], (((1,), (1,)), ((), ())),
                        preferred_element_type=jnp.float32)
    h = h + lax.dot_general(im, w1b_ref[...], (((1,), (1,)), ((), ())),
                            preferred_element_type=jnp.float32)
    h = jnp.maximum(h + b1_ref[...], 0.0)
    o_ref[...] = lax.dot_general(h, w2_ref[...], (((1,), (0,)), ((), ())),
                                 preferred_element_type=jnp.float32) + b2_ref[0]


def _tc_mlp(u_wide, i_wide, u_rem, i_rem, w1a, w1b, b1, W2, b2):
    blk = 2048
    grid = (B // blk,)
    return pl.pallas_call(
        _mlp_body,
        grid=grid,
        in_specs=[
            pl.BlockSpec((blk, 128), lambda b: (b, 0)),
            pl.BlockSpec((blk, 128), lambda b: (b, 0)),
            pl.BlockSpec((blk, 1), lambda b: (b, 0)),
            pl.BlockSpec((blk, 1), lambda b: (b, 0)),
            pl.BlockSpec((128, 128), lambda b: (0, 0)),
            pl.BlockSpec((128, 128), lambda b: (0, 0)),
            pl.BlockSpec((1, 128), lambda b: (0, 0)),
            pl.BlockSpec((128, 1), lambda b: (0, 0)),
            pl.BlockSpec(memory_space=pltpu.SMEM),
        ],
        out_specs=pl.BlockSpec((blk, 1), lambda b: (b, 0)),
        out_shape=jax.ShapeDtypeStruct((B, 1), jnp.float32),
    )(u_wide, i_wide, u_rem, i_rem, w1a, w1b, b1, W2, b2)


def kernel(user, item, user_table, item_table, W1, b1, W2, b2):
    user = user.astype(jnp.int32)
    item = item.astype(jnp.int32)
    ut4, it4 = _tc_relayout(user_table.T, item_table.T)
    u_idx4 = (user % S).reshape(NW, NCHUNK, CHUNK)
    i_idx4 = (item % S).reshape(NW, NCHUNK, CHUNK)
    u_wide, i_wide = _sc_gather(ut4, it4, u_idx4, i_idx4)
    w1a = jnp.tile(W1[:, :D], (1, GRP))
    w1b = jnp.tile(W1[:, D:], (1, GRP))
    out = _tc_mlp(u_wide, i_wide,
                  (user // S).astype(jnp.float32).reshape(B, 1),
                  (item // S).astype(jnp.float32).reshape(B, 1),
                  w1a, w1b,
                  b1.reshape(1, 128), W2.reshape(128, 1), b2)
    return out[:, 0]


# speedup vs baseline: 1.0012x; 1.0012x over previous
import functools

import jax
import jax.numpy as jnp
from jax import lax
from jax.experimental import pallas as pl
from jax.experimental.pallas import tpu as pltpu
from jax.experimental.pallas import tpu_sc as plsc

NC, NS = 2, 16
NW = NC * NS
B = 16384
D = 32
GRP = 128 // D
BPW = B // NW
CHUNK = 128
NCHUNK = BPW // CHUNK
RL = 4096
S = 131072
NB = S // RL


def _bf16_top(x):
    xi = lax.bitcast_convert_type(x, jnp.uint32)
    r = xi + jnp.uint32(0x7FFF) + ((xi >> 16) & jnp.uint32(1))
    return r & jnp.uint32(0xFFFF0000)


def _relayout_body(*refs):
    uo_ref, io_ref = refs[16], refs[17]
    for out_ref, blocks in ((uo_ref, refs[0:8]), (io_ref, refs[8:16])):
        lo = jnp.concatenate([b[...] for b in blocks[:4]], axis=0)
        hi = jnp.concatenate([b[...] for b in blocks[4:]], axis=0)
        w = _bf16_top(lo) | (_bf16_top(hi) >> 16)
        out_ref[...] = lax.bitcast_convert_type(w.T, jnp.float32)


def _tc_relayout(ut_t, it_t):
    nvb = (ut_t.shape[1] + RL - 1) // RL
    specs = [pl.BlockSpec(
                 (32, RL),
                 (lambda p: (lambda b: (0, jnp.minimum(p * NB + b, nvb - 1))))(p))
             for p in range(8)]
    mesh = pltpu.create_tensorcore_mesh("tc", num_cores=1)

    @functools.partial(
        pl.kernel,
        mesh=mesh,
        out_type=[jax.ShapeDtypeStruct((S, 128), jnp.float32),
                  jax.ShapeDtypeStruct((S, 128), jnp.float32)],
    )
    def relayout_kernel(ut_hbm, it_hbm, uo_hbm, io_hbm):
        pltpu.emit_pipeline(
            _relayout_body,
            grid=(NB,),
            in_specs=[pl.BlockSpec((32, RL), s.index_map) for s in specs] * 2,
            out_specs=[pl.BlockSpec((RL, 128), lambda b: (b, 0)),
                       pl.BlockSpec((RL, 128), lambda b: (b, 0))],
            core_axis_name="tc",
            dimension_semantics=(pltpu.PARALLEL,),
        )(*([ut_hbm] * 8 + [it_hbm] * 8 + [uo_hbm, io_hbm]))

    return relayout_kernel(ut_t, it_t)


def _sc_gather(ut4, it4, u_idx4, i_idx4):
    mesh = plsc.VectorSubcoreMesh(core_axis_name="c", subcore_axis_name="s")

    @functools.partial(
        pl.kernel,
        mesh=mesh,
        out_type=[
            jax.ShapeDtypeStruct((B, 128), jnp.float32),
            jax.ShapeDtypeStruct((B, 128), jnp.float32),
        ],
        scratch_types=[
            pltpu.VMEM((NCHUNK, CHUNK), jnp.int32),
            pltpu.VMEM((NCHUNK, CHUNK), jnp.int32),
            pltpu.VMEM((4, CHUNK, 128), jnp.float32),
            pltpu.SemaphoreType.DMA,
            pltpu.SemaphoreType.DMA,
        ],
    )
    def gather_kernel(ut_hbm, it_hbm, ui_hbm, ii_hbm, u_out, i_out,
                      uidx_v, iidx_v, bufs, gsem, wsem):
        wid = lax.axis_index("s") * NC + lax.axis_index("c")
        base = wid * BPW
        pltpu.sync_copy(ui_hbm.at[wid], uidx_v)
        pltpu.sync_copy(ii_hbm.at[wid], iidx_v)
        for tab, idx, out in ((ut_hbm, uidx_v, u_out), (it_hbm, iidx_v, i_out)):
            gds = [pltpu.async_copy(tab.at[idx.at[c]], bufs.at[c], gsem)
                   for c in range(NCHUNK)]
            for g in gds:
                g.wait()
            wbs = [pltpu.async_copy(
                       bufs.at[c], out.at[pl.ds(base + c * CHUNK, CHUNK)], wsem)
                   for c in range(NCHUNK)]
            for w in wbs:
                w.wait()

    return gather_kernel(ut4, it4, u_idx4, i_idx4)


def _mlp_body(uw_ref, iw_ref, ur_ref, ir_ref,
              w1a_ref, w1b_ref, b1_ref, w2_ref, b2_ref, o_ref):
    blk = uw_ref.shape[0]
    ones = jnp.ones((1, 128), jnp.float32)
    urb = lax.dot_general(ur_ref[...], ones, (((1,), (0,)), ((), ())),
                          preferred_element_type=jnp.float32)
    irb = lax.dot_general(ir_ref[...], ones, (((1,), (0,)), ((), ())),
                          preferred_element_type=jnp.float32)
    group = (lax.broadcasted_iota(jnp.int32, (blk, 128), 1) // D
             ).astype(jnp.float32)
    uw = lax.bitcast_convert_type(uw_ref[...], jnp.uint32)
    iw = lax.bitcast_convert_type(iw_ref[...], jnp.uint32)
    uv = lax.bitcast_convert_type(
        jnp.where(urb >= 4.0, uw << 16, uw & jnp.uint32(0xFFFF0000)),
        jnp.float32)
    iv = lax.bitcast_convert_type(
        jnp.where(irb >= 4.0, iw << 16, iw & jnp.uint32(0xFFFF0000)),
        jnp.float32)
    upm = urb - 4.0 * jnp.floor(urb * 0.25)
    ipm = irb - 4.0 * jnp.floor(irb * 0.25)
    um = jnp.where(upm == group, uv, 0.0)
    im = jnp.where(ipm == group, iv, 0.0)
    h = lax.dot_general(um, w1a_ref[...], (((1,), (1,)), ((), ())),
                        preferred_element_type=jnp.float32)
    h = h + lax.dot_general(im, w1b_ref[...], (((1,), (1,)), ((), ())),
                            preferred_element_type=jnp.float32)
    h = jnp.maximum(h + b1_ref[...], 0.0)
    o_ref[...] = lax.dot_general(h, w2_ref[...], (((1,), (0,)), ((), ())),
                                 preferred_element_type=jnp.float32) + b2_ref[0]


def _tc_mlp(u_wide, i_wide, u_rem, i_rem, w1a, w1b, b1, W2, b2):
    blk = 2048
    grid = (B // blk,)
    return pl.pallas_call(
        _mlp_body,
        grid=grid,
        in_specs=[
            pl.BlockSpec((blk, 128), lambda b: (b, 0)),
            pl.BlockSpec((blk, 128), lambda b: (b, 0)),
            pl.BlockSpec((blk, 1), lambda b: (b, 0)),
            pl.BlockSpec((blk, 1), lambda b: (b, 0)),
            pl.BlockSpec((128, 128), lambda b: (0, 0)),
            pl.BlockSpec((128, 128), lambda b: (0, 0)),
            pl.BlockSpec((1, 128), lambda b: (0, 0)),
            pl.BlockSpec((128, 1), lambda b: (0, 0)),
            pl.BlockSpec(memory_space=pltpu.SMEM),
        ],
        out_specs=pl.BlockSpec((blk, 1), lambda b: (b, 0)),
        out_shape=jax.ShapeDtypeStruct((B, 1), jnp.float32),
    )(u_wide, i_wide, u_rem, i_rem, w1a, w1b, b1, W2, b2)


def kernel(user, item, user_table, item_table, W1, b1, W2, b2):
    user = user.astype(jnp.int32)
    item = item.astype(jnp.int32)
    ut4, it4 = _tc_relayout(user_table.T, item_table.T)
    u_idx4 = (user % S).reshape(NW, NCHUNK, CHUNK)
    i_idx4 = (item % S).reshape(NW, NCHUNK, CHUNK)
    u_wide, i_wide = _sc_gather(ut4, it4, u_idx4, i_idx4)
    w1a = jnp.tile(W1[:, :D], (1, GRP))
    w1b = jnp.tile(W1[:, D:], (1, GRP))
    out = _tc_mlp(u_wide, i_wide,
                  (user // S).astype(jnp.float32).reshape(B, 1),
                  (item // S).astype(jnp.float32).reshape(B, 1),
                  w1a, w1b,
                  b1.reshape(1, 128), W2.reshape(128, 1), b2)
    return out[:, 0]


# speedup vs baseline: 1.0023x; 1.0012x over previous
import functools

import jax
import jax.numpy as jnp
from jax import lax
from jax.experimental import pallas as pl
from jax.experimental.pallas import tpu as pltpu
from jax.experimental.pallas import tpu_sc as plsc

NC, NS = 2, 16
NW = NC * NS
B = 16384
D = 32
GRP = 128 // D
BPW = B // NW
CHUNK = 128
NCHUNK = BPW // CHUNK
RL = 4096
S = 131072
NB = S // RL


def _bf16_top(x):
    xi = lax.bitcast_convert_type(x, jnp.uint32)
    r = xi + jnp.uint32(0x7FFF) + ((xi >> 16) & jnp.uint32(1))
    return r & jnp.uint32(0xFFFF0000)


def _relayout_body(*refs):
    uo_ref, io_ref = refs[16], refs[17]
    for out_ref, blocks in ((uo_ref, refs[0:8]), (io_ref, refs[8:16])):
        lo = jnp.concatenate([b[...] for b in blocks[:4]], axis=0)
        hi = jnp.concatenate([b[...] for b in blocks[4:]], axis=0)
        w = _bf16_top(lo) | (_bf16_top(hi) >> 16)
        out_ref[...] = lax.bitcast_convert_type(w.T, jnp.float32)


def _tc_relayout(ut_t, it_t):
    nvb = (ut_t.shape[1] + RL - 1) // RL
    specs = [pl.BlockSpec(
                 (32, RL),
                 (lambda p: (lambda b: (0, jnp.minimum(p * NB + b, nvb - 1))))(p))
             for p in range(8)]
    return pl.pallas_call(
        _relayout_body,
        grid=(NB,),
        in_specs=specs + [pl.BlockSpec((32, RL), s.index_map) for s in specs],
        out_specs=[pl.BlockSpec((RL, 128), lambda b: (b, 0)),
                   pl.BlockSpec((RL, 128), lambda b: (b, 0))],
        out_shape=[jax.ShapeDtypeStruct((S, 128), jnp.float32),
                   jax.ShapeDtypeStruct((S, 128), jnp.float32)],
        compiler_params=pltpu.CompilerParams(
            dimension_semantics=(pltpu.PARALLEL,)),
    )(*([ut_t] * 8 + [it_t] * 8))


def _sc_gather(ut4, it4, u_idx4, i_idx4):
    mesh = plsc.VectorSubcoreMesh(core_axis_name="c", subcore_axis_name="s")

    @functools.partial(
        pl.kernel,
        mesh=mesh,
        out_type=[
            jax.ShapeDtypeStruct((B, 128), jnp.float32),
            jax.ShapeDtypeStruct((B, 128), jnp.float32),
        ],
        scratch_types=[
            pltpu.VMEM((NCHUNK, CHUNK), jnp.int32),
            pltpu.VMEM((NCHUNK, CHUNK), jnp.int32),
            pltpu.VMEM((4, CHUNK, 128), jnp.float32),
            pltpu.SemaphoreType.DMA,
            pltpu.SemaphoreType.DMA,
        ],
    )
    def gather_kernel(ut_hbm, it_hbm, ui_hbm, ii_hbm, u_out, i_out,
                      uidx_v, iidx_v, bufs, gsem, wsem):
        wid = lax.axis_index("s") * NC + lax.axis_index("c")
        base = wid * BPW
        pltpu.sync_copy(ui_hbm.at[wid], uidx_v)
        pltpu.sync_copy(ii_hbm.at[wid], iidx_v)
        for tab, idx, out in ((ut_hbm, uidx_v, u_out), (it_hbm, iidx_v, i_out)):
            gds = [pltpu.async_copy(tab.at[idx.at[c]], bufs.at[c], gsem)
                   for c in range(NCHUNK)]
            for g in gds:
                g.wait()
            wbs = [pltpu.async_copy(
                       bufs.at[c], out.at[pl.ds(base + c * CHUNK, CHUNK)], wsem)
                   for c in range(NCHUNK)]
            for w in wbs:
                w.wait()

    return gather_kernel(ut4, it4, u_idx4, i_idx4)


def _mlp_body(uw_ref, iw_ref, ur_ref, ir_ref,
              w1a_ref, w1b_ref, b1_ref, w2_ref, b2_ref, o_ref):
    blk = uw_ref.shape[0]
    ones = jnp.ones((1, 128), jnp.float32)
    urb = lax.dot_general(ur_ref[...], ones, (((1,), (0,)), ((), ())),
                          preferred_element_type=jnp.float32)
    irb = lax.dot_general(ir_ref[...], ones, (((1,), (0,)), ((), ())),
                          preferred_element_type=jnp.float32)
    group = (lax.broadcasted_iota(jnp.int32, (blk, 128), 1) // D
             ).astype(jnp.float32)
    uw = lax.bitcast_convert_type(uw_ref[...], jnp.uint32)
    iw = lax.bitcast_convert_type(iw_ref[...], jnp.uint32)
    uv = lax.bitcast_convert_type(
        jnp.where(urb >= 4.0, uw << 16, uw & jnp.uint32(0xFFFF0000)),
        jnp.float32)
    iv = lax.bitcast_convert_type(
        jnp.where(irb >= 4.0, iw << 16, iw & jnp.uint32(0xFFFF0000)),
        jnp.float32)
    upm = urb - 4.0 * jnp.floor(urb * 0.25)
    ipm = irb - 4.0 * jnp.floor(irb * 0.25)
    um = jnp.where(upm == group, uv, 0.0)
    im = jnp.where(ipm == group, iv, 0.0)
    h = lax.dot_general(um, w1a_ref[...], (((1,), (1,)), ((), ())),
                        preferred_element_type=jnp.float32)
    h = h + lax.dot_general(im, w1b_ref[...], (((1,), (1,)), ((), ())),
                            preferred_element_type=jnp.float32)
    h = jnp.maximum(h + b1_ref[...], 0.0)
    o_ref[...] = lax.dot_general(h, w2_ref[...], (((1,), (0,)), ((), ())),
                                 preferred_element_type=jnp.float32) + b2_ref[0]


def _tc_mlp(u_wide, i_wide, u_rem, i_rem, w1a, w1b, b1, W2, b2):
    blk = 2048
    grid = (B // blk,)
    return pl.pallas_call(
        _mlp_body,
        grid=grid,
        in_specs=[
            pl.BlockSpec((blk, 128), lambda b: (b, 0)),
            pl.BlockSpec((blk, 128), lambda b: (b, 0)),
            pl.BlockSpec((blk, 1), lambda b: (b, 0)),
            pl.BlockSpec((blk, 1), lambda b: (b, 0)),
            pl.BlockSpec((128, 128), lambda b: (0, 0)),
            pl.BlockSpec((128, 128), lambda b: (0, 0)),
            pl.BlockSpec((1, 128), lambda b: (0, 0)),
            pl.BlockSpec((128, 1), lambda b: (0, 0)),
            pl.BlockSpec(memory_space=pltpu.SMEM),
        ],
        out_specs=pl.BlockSpec((blk, 1), lambda b: (b, 0)),
        out_shape=jax.ShapeDtypeStruct((B, 1), jnp.float32),
    )(u_wide, i_wide, u_rem, i_rem, w1a, w1b, b1, W2, b2)


def kernel(user, item, user_table, item_table, W1, b1, W2, b2):
    user = user.astype(jnp.int32)
    item = item.astype(jnp.int32)
    ut4, it4 = _tc_relayout(user_table.T, item_table.T)
    u_idx4 = (user % S).reshape(NW, NCHUNK, CHUNK)
    i_idx4 = (item % S).reshape(NW, NCHUNK, CHUNK)
    u_wide, i_wide = _sc_gather(ut4, it4, u_idx4, i_idx4)
    w1a = jnp.tile(W1[:, :D], (1, GRP))
    w1b = jnp.tile(W1[:, D:], (1, GRP))
    out = _tc_mlp(u_wide, i_wide,
                  (user // S).astype(jnp.float32).reshape(B, 1),
                  (item // S).astype(jnp.float32).reshape(B, 1),
                  w1a, w1b,
                  b1.reshape(1, 128), W2.reshape(128, 1), b2)
    return out[:, 0]
